# Initial kernel scaffold; baseline (speedup 1.0000x reference)
import jax
import jax.numpy as jnp
from jax.experimental import pallas as pl


def kernel(x_node, edge_index_0, edge_index_1, conv0_wl_w, conv0_wl_b, conv0_w0_w, conv0_w0_b, conv0_w1_w, conv0_w1_b, conv1_wl_w, conv1_wl_b, conv1_w0_w, conv1_w0_b, conv1_w1_w, conv1_w1_b, out_w, out_b):
    raise NotImplementedError("write your pallas kernel here")


# speedup vs baseline: 156.0257x; 156.0257x over previous
import functools

import jax
import jax.numpy as jnp
from jax import lax
from jax.experimental import pallas as pl
from jax.experimental.pallas import tpu as pltpu
from jax.experimental.pallas import tpu_sc as plsc

N = 10000
NP = 10240
C = 128
E = 320000
NC = 2
NS = 16
NW = NC * NS
EPW = E // NW
CHUNK = 80
NCHUNK = EPW // CHUNK
RPT = NP // NS


def _mesh():
    return plsc.VectorSubcoreMesh(core_axis_name="c", subcore_axis_name="s")


@functools.partial(
    pl.kernel,
    out_type=jax.ShapeDtypeStruct((NC, 2, NP), jnp.int32),
    mesh=_mesh(),
    compiler_params=pltpu.CompilerParams(needs_layout_passes=False),
    scratch_types=[
        pltpu.VMEM_SHARED((NS, 2, NP), jnp.int32),
        pltpu.VMEM((NP,), jnp.int32),
        pltpu.VMEM((NP,), jnp.int32),
        pltpu.VMEM((EPW,), jnp.int32),
        pltpu.VMEM((EPW,), jnp.int32),
        pltpu.VMEM((RPT,), jnp.int32),
        pltpu.VMEM((RPT,), jnp.int32),
        pltpu.VMEM((RPT,), jnp.int32),
    ],
)
def _count(e0, e1, out, stage, cnt0, cnt1, e0b, e1b, tmp, acc0, acc1):
    c = lax.axis_index("c")
    s = lax.axis_index("s")
    base = (c * NS + s) * EPW
    pltpu.sync_copy(e0.at[pl.ds(base, EPW)], e0b)
    pltpu.sync_copy(e1.at[pl.ds(base, EPW)], e1b)

    z16 = jnp.zeros((16,), jnp.int32)

    def zero_cnt(i, _):
        cnt0[pl.ds(i * 16, 16)] = z16
        cnt1[pl.ds(i * 16, 16)] = z16
        return 0

    lax.fori_loop(0, NP // 16, zero_cnt, 0)

    ones = jnp.ones((16,), jnp.int32)

    def scat(i, _):
        plsc.addupdate_scatter(cnt0, [e0b[pl.ds(i * 16, 16)]], ones)
        plsc.addupdate_scatter(cnt1, [e1b[pl.ds(i * 16, 16)]], ones)
        return 0

    lax.fori_loop(0, EPW // 16, scat, 0)

    pltpu.sync_copy(cnt0, stage.at[s, 0])
    pltpu.sync_copy(cnt1, stage.at[s, 1])
    plsc.subcore_barrier()

    rbase = s * RPT

    def zero_acc(i, _):
        acc0[pl.ds(i * 16, 16)] = z16
        acc1[pl.ds(i * 16, 16)] = z16
        return 0

    lax.fori_loop(0, RPT // 16, zero_acc, 0)

    for j in range(NS):
        pltpu.sync_copy(stage.at[j, 0, pl.ds(rbase, RPT)], tmp)

        def add0(i, _):
            acc0[pl.ds(i * 16, 16)] = acc0[pl.ds(i * 16, 16)] + tmp[pl.ds(i * 16, 16)]
            return 0

        lax.fori_loop(0, RPT // 16, add0, 0)
        pltpu.sync_copy(stage.at[j, 1, pl.ds(rbase, RPT)], tmp)

        def add1(i, _):
            acc1[pl.ds(i * 16, 16)] = acc1[pl.ds(i * 16, 16)] + tmp[pl.ds(i * 16, 16)]
            return 0

        lax.fori_loop(0, RPT // 16, add1, 0)

    pltpu.sync_copy(acc0, out.at[c, 0, pl.ds(rbase, RPT)])
    pltpu.sync_copy(acc1, out.at[c, 1, pl.ds(rbase, RPT)])


@functools.partial(
    pl.kernel,
    out_type=jax.ShapeDtypeStruct((NC, NP, C), jnp.float32),
    mesh=_mesh(),
    compiler_params=pltpu.CompilerParams(needs_layout_passes=False),
    scratch_types=[
        pltpu.VMEM_SHARED((NP, C), jnp.float32),
        pltpu.VMEM((NP,), jnp.int32),
        pltpu.VMEM((EPW,), jnp.int32),
        pltpu.VMEM((EPW,), jnp.int32),
        pltpu.VMEM((1, CHUNK), jnp.int32),
        pltpu.VMEM((CHUNK, C), jnp.float32),
        pltpu.SemaphoreType.DMA,
    ],
)
def _agg(h, e0, e1, tmap_h, out, agg_sh, tmap, e0b, e1b, idxT, rows, sem):
    c = lax.axis_index("c")
    s = lax.axis_index("s")
    base = (c * NS + s) * EPW
    pltpu.sync_copy(tmap_h, tmap)
    pltpu.sync_copy(e0.at[pl.ds(base, EPW)], e0b)
    pltpu.sync_copy(e1.at[pl.ds(base, EPW)], e1b)

    zf = jnp.zeros((16,), jnp.float32)

    def zrow(i, _):
        for k in range(C // 16):
            rows[i, pl.ds(k * 16, 16)] = zf
        return 0

    lax.fori_loop(0, CHUNK, zrow, 0)
    for k in range(RPT // CHUNK):
        pltpu.sync_copy(rows, agg_sh.at[pl.ds(s * RPT + k * CHUNK, CHUNK)])
    plsc.subcore_barrier()

    def chunk(i, _):
        off = i * CHUNK
        for k in range(CHUNK // 16):
            idxT[0, pl.ds(k * 16, 16)] = plsc.load_gather(
                tmap, [e0b[pl.ds(off + k * 16, 16)]])
        pltpu.async_copy(h.at[e1b.at[pl.ds(off, CHUNK)]], rows, sem).wait()
        pltpu.sync_copy(rows, agg_sh.at[idxT.at[0]], add=True)
        return 0

    lax.fori_loop(0, NCHUNK, chunk, 0)

    plsc.subcore_barrier()
    pltpu.sync_copy(agg_sh.at[pl.ds(s * RPT, RPT)], out.at[c, pl.ds(s * RPT, RPT)])


def _fuse_body(a0, a1, h, m, wl, w01, b, o):
    z = jnp.dot(a0[...] + a1[...], wl[...], preferred_element_type=jnp.float32)
    z = z + jnp.dot(h[...], w01[...], preferred_element_type=jnp.float32)
    z = z + b[...]
    o[...] = jnp.where(m[...] > 0.0, jnp.maximum(z, 0.0), h[...])


_R = 1024
_fuse_call = pl.pallas_call(
    _fuse_body,
    grid=(NP // _R,),
    in_specs=[
        pl.BlockSpec((_R, C), lambda i: (i, 0)),
        pl.BlockSpec((_R, C), lambda i: (i, 0)),
        pl.BlockSpec((_R, C), lambda i: (i, 0)),
        pl.BlockSpec((_R, 1), lambda i: (i, 0)),
        pl.BlockSpec((C, C), lambda i: (0, 0)),
        pl.BlockSpec((C, C), lambda i: (0, 0)),
        pl.BlockSpec((1, C), lambda i: (0, 0)),
    ],
    out_specs=pl.BlockSpec((_R, C), lambda i: (i, 0)),
    out_shape=jax.ShapeDtypeStruct((NP, C), jnp.float32),
)


def _out_body(h, w, b, o):
    o[...] = jnp.dot(h[...], w[...], preferred_element_type=jnp.float32) + b[...]


_out_call = pl.pallas_call(
    _out_body,
    grid=(NP // _R,),
    in_specs=[
        pl.BlockSpec((_R, C), lambda i: (i, 0)),
        pl.BlockSpec((C, C), lambda i: (0, 0)),
        pl.BlockSpec((1, C), lambda i: (0, 0)),
    ],
    out_specs=pl.BlockSpec((_R, C), lambda i: (i, 0)),
    out_shape=jax.ShapeDtypeStruct((NP, C), jnp.float32),
)


def _relation(h, edge, wl_w, wl_b, w0_w, w0_b, w1_w, w1_b):
    e0 = edge[0]
    e1 = edge[1]
    cnts = _count(e0, e1)
    cnt_src = cnts[0, 0] + cnts[1, 0]
    cnt_dst = cnts[0, 1] + cnts[1, 1]
    psrc = (cnt_src > 0).astype(jnp.int32)
    pdst = (cnt_dst > 0).astype(jnp.int32)
    rank_src = jnp.cumsum(psrc) - psrc
    rank_dst = jnp.cumsum(pdst) - pdst
    dlist = jnp.full((NP,), N, jnp.int32).at[
        jnp.where(pdst > 0, rank_dst, NP)
    ].set(jnp.arange(NP, dtype=jnp.int32), mode="drop")
    tmap = dlist[rank_src]
    aggs = _agg(h, e0, e1, tmap)
    mask = pdst.astype(jnp.float32).reshape(NP, 1)
    return _fuse_call(aggs[0], aggs[1], h, mask, wl_w.T, (w0_w + w1_w).T,
                      (wl_b + w0_b + w1_b).reshape(1, C))


def kernel(x_node, edge_index_0, edge_index_1,
           conv0_wl_w, conv0_wl_b, conv0_w0_w, conv0_w0_b, conv0_w1_w, conv0_w1_b,
           conv1_wl_w, conv1_wl_b, conv1_w0_w, conv1_w0_b, conv1_w1_w, conv1_w1_b,
           out_w, out_b):
    h = jnp.pad(x_node, ((0, NP - N), (0, 0)))
    h = _relation(h, edge_index_1, conv1_wl_w, conv1_wl_b,
                  conv1_w0_w, conv1_w0_b, conv1_w1_w, conv1_w1_b)
    h = _relation(h, edge_index_0, conv0_wl_w, conv0_wl_b,
                  conv0_w0_w, conv0_w0_b, conv0_w1_w, conv0_w1_b)
    out = _out_call(h, out_w.T, out_b.reshape(1, C))
    return out[:N]
